# Initial kernel scaffold; baseline (speedup 1.0000x reference)
import jax
import jax.numpy as jnp
from jax.experimental import pallas as pl


def kernel(rpn_bbox_deltas, rpn_labels, anchors):
    raise NotImplementedError("write your pallas kernel here")


# speedup vs baseline: 16.6196x; 16.6196x over previous
import functools

import jax
import jax.numpy as jnp
from jax import lax
from jax.experimental import pallas as pl
from jax.experimental.pallas import tpu as pltpu
from jax.experimental.pallas import tpu_sc as plsc

B = 8
A = 20000
AP = 20480
K = 2000
KP = 2048
OUTN = 300
OUTP = 384
IOU_T = 0.7
NEG_INF = float("-inf")


def _select_body(lab_ref, key_ref, aux_ref, md_ref):
    x = lab_ref[...]
    m = jnp.max(x, axis=1, keepdims=True)
    u = jnp.exp(x - m)
    d = jnp.sum(u, axis=1, keepdims=True)
    p = u / d
    key = lax.bitcast_convert_type(p, jnp.int32)
    col = lax.broadcasted_iota(jnp.int32, (B, AP), 1)
    key = jnp.where(col < A, key, -1)

    def bit_step(k, t):
        bit = 30 - k
        cand = t | jnp.left_shift(jnp.int32(1), bit)
        cnt = jnp.sum((key >= cand).astype(jnp.int32), axis=1, keepdims=True)
        return jnp.where(cnt >= K, cand, t)

    t = lax.fori_loop(0, 31, bit_step, jnp.zeros((B, 1), jnp.int32))
    c1 = jnp.sum((key > t).astype(jnp.int32), axis=1, keepdims=True)
    k2 = K - c1
    lane = lax.broadcasted_iota(jnp.int32, (B, 128), 1)
    aux_ref[...] = jnp.where(lane < 16, t, jnp.where(lane < 32, k2, 0))
    md_ref[...] = jnp.where(lane == 0, m, jnp.where(lane == 1, d, 0.0))
    key_ref[...] = key


def _run_select(labels_p):
    return pl.pallas_call(
        _select_body,
        out_shape=(
            jax.ShapeDtypeStruct((B, AP), jnp.int32),
            jax.ShapeDtypeStruct((B, 128), jnp.int32),
            jax.ShapeDtypeStruct((B, 128), jnp.float32),
        ),
    )(labels_p)


_CH = 2560


def _decode_body(dt_ref, lab_ref, at_ref, md_ref, out_ref):
    x = lab_ref[...]
    m = md_ref[:, 0:1]
    den = md_ref[:, 1:2]
    p = jnp.exp(x - m) / den
    d0 = dt_ref[:, 0, :] * jnp.float32(0.1)
    d1 = dt_ref[:, 1, :] * jnp.float32(0.1)
    d2 = dt_ref[:, 2, :] * jnp.float32(0.2)
    d3 = dt_ref[:, 3, :] * jnp.float32(0.2)
    a0 = at_ref[0:1, :]
    a1 = at_ref[1:2, :]
    a2 = at_ref[2:3, :]
    a3 = at_ref[3:4, :]
    w = a3 - a1
    h = a2 - a0
    cx = a1 + 0.5 * w
    cy = a0 + 0.5 * h
    bw = jnp.exp(d3) * w
    bh = jnp.exp(d2) * h
    bcx = d1 * w + cx
    bcy = d0 * h + cy
    y1 = bcy - 0.5 * bh
    x1 = bcx - 0.5 * bw
    y2 = bh + y1
    x2 = bw + x1
    out_ref[0] = y1
    out_ref[1] = x1
    out_ref[2] = y2
    out_ref[3] = x2
    out_ref[4] = p


def _run_decode(deltas_t, labels_p, anchors_t, md):
    n_ch = AP // _CH
    return pl.pallas_call(
        _decode_body,
        grid=(n_ch,),
        in_specs=[
            pl.BlockSpec((B, 4, _CH), lambda i: (0, 0, i)),
            pl.BlockSpec((B, _CH), lambda i: (0, i)),
            pl.BlockSpec((4, _CH), lambda i: (0, i)),
            pl.BlockSpec((B, 128), lambda i: (0, 0)),
        ],
        out_specs=pl.BlockSpec((5, B, _CH), lambda i: (0, 0, i)),
        out_shape=jax.ShapeDtypeStruct((5, B, AP), jnp.float32),
    )(deltas_t, labels_p, anchors_t, md)


_NPB = 4
_SEG = AP // _NPB
_NCHUNK = _SEG // 16


def _sc_body(key_hbm, aux_hbm, p0_hbm, p1_hbm, p2_hbm, p3_hbm, p4_hbm,
             gath_hbm,
             key_v, aux_v, idx_buf, aidx_v, gidx_v, row_v, cnt_tmp,
             cnt_gt_sp, cnt_eq_sp, merged_sp, sem):
    c = lax.axis_index("c")
    s = lax.axis_index("s")
    b = c * 4 + s // _NPB
    bc = s // _NPB
    q = s % _NPB
    base = q * _SEG

    pltpu.sync_copy(key_hbm.at[pl.ds(b * AP + base, _SEG)], key_v)
    pltpu.sync_copy(aux_hbm.at[pl.ds(b * 128, 32)], aux_v)
    tvec = aux_v[pl.ds(0, 16)]
    k2vec = aux_v[pl.ds(16, 16)]

    def cnt_step(i, carry):
        cgt, ceq = carry
        kv = key_v[pl.ds(i * 16, 16)]
        cgt = cgt + plsc.all_reduce_population_count(kv > tvec)
        ceq = ceq + plsc.all_reduce_population_count(kv == tvec)
        return cgt, ceq

    zero16 = jnp.zeros((16,), jnp.int32)
    cgt, ceq = lax.fori_loop(0, _NCHUNK, cnt_step, (zero16, zero16))
    cnt_tmp[...] = cgt
    pltpu.sync_copy(cnt_tmp, cnt_gt_sp.at[s])
    cnt_tmp[...] = ceq
    pltpu.sync_copy(cnt_tmp, cnt_eq_sp.at[s])

    lane = lax.iota(jnp.int32, 16)
    abase = bc * KP

    def zero_step(j, _):
        idx_buf[pl.ds(j * 16, 16)] = zero16
        aidx_v[pl.ds(j * 16, 16)] = lane + (abase + j * 16)
        return 0

    lax.fori_loop(0, KP // 16, zero_step, 0)

    @pl.when(q == 0)
    def _():
        pltpu.sync_copy(idx_buf, merged_sp.at[pl.ds(abase, KP)])

    plsc.subcore_barrier()

    carry_gt = zero16
    carry_eq = zero16
    for qq in range(_NPB - 1):
        r = s - q + qq
        take = (zero16 + qq) < (zero16 + q)
        pltpu.sync_copy(cnt_gt_sp.at[r], cnt_tmp)
        carry_gt = carry_gt + jnp.where(take, cnt_tmp[...], 0)
        pltpu.sync_copy(cnt_eq_sp.at[r], cnt_tmp)
        carry_eq = carry_eq + jnp.where(take, cnt_tmp[...], 0)
    carry_sel = carry_gt + jnp.minimum(carry_eq, k2vec)

    gbase = b * AP + base + 1

    def scan_step(i, carry):
        carry_eq, carry_sel = carry
        kv = key_v[pl.ds(i * 16, 16)]
        gt = kv > tvec
        eq = kv == tvec
        eq_i = eq.astype(jnp.int32)
        pref_eq = carry_eq + plsc.cumsum(eq_i)
        sel = gt | (eq & (pref_eq <= k2vec))
        sel_i = sel.astype(jnp.int32)
        incl = plsc.cumsum(sel_i)
        pos = carry_sel + incl - sel_i
        vals = lane + (gbase + i * 16)
        plsc.store_scatter(idx_buf, [pos], vals, mask=sel)
        carry_eq = carry_eq + plsc.all_reduce_population_count(eq)
        carry_sel = carry_sel + plsc.all_reduce_population_count(sel)
        return carry_eq, carry_sel

    lax.fori_loop(0, _NCHUNK, scan_step, (carry_eq, carry_sel))
    pltpu.sync_copy(idx_buf, merged_sp.at[aidx_v], add=True)
    plsc.subcore_barrier()

    nsl = KP // _NPB
    pltpu.sync_copy(merged_sp.at[pl.ds(abase + q * nsl, nsl)], gidx_v)

    def adj_step(j, _):
        v = gidx_v[pl.ds(j * 16, 16)]
        gidx_v[pl.ds(j * 16, 16)] = jnp.maximum(v - 1, 0)
        return 0

    lax.fori_loop(0, nsl // 16, adj_step, 0)
    off = b * KP + q * nsl
    for ci, plane in enumerate((p0_hbm, p1_hbm, p2_hbm, p3_hbm, p4_hbm)):
        pltpu.async_copy(plane.at[gidx_v], row_v, sem).wait()
        pltpu.sync_copy(row_v, gath_hbm.at[pl.ds(ci * (B * KP) + off, nsl)])


def _run_sc(key, aux, planes_flat):
    mesh = plsc.VectorSubcoreMesh(core_axis_name="c", subcore_axis_name="s")
    nsl = KP // _NPB
    fn = pl.kernel(
        _sc_body,
        out_type=jax.ShapeDtypeStruct((5 * B * KP,), jnp.float32),
        mesh=mesh,
        scratch_types=[
            pltpu.VMEM((_SEG,), jnp.int32),
            pltpu.VMEM((32,), jnp.int32),
            pltpu.VMEM((KP,), jnp.int32),
            pltpu.VMEM((KP,), jnp.int32),
            pltpu.VMEM((nsl,), jnp.int32),
            pltpu.VMEM((nsl,), jnp.float32),
            pltpu.VMEM((16,), jnp.int32),
            pltpu.VMEM_SHARED((16, 16), jnp.int32),
            pltpu.VMEM_SHARED((16, 16), jnp.int32),
            pltpu.VMEM_SHARED((4 * KP,), jnp.int32),
            pltpu.SemaphoreType.DMA,
        ],
        compiler_params=pltpu.CompilerParams(needs_layout_passes=False),
    )
    return fn(key.reshape(-1), aux.reshape(-1), planes_flat[0],
              planes_flat[1], planes_flat[2], planes_flat[3], planes_flat[4])


_TILE = 256


def _nms_body(pr_ref, pc_ref, boxes_ref, scores_ref, s_ref):
    y1r = pr_ref[0, 0:1, :]
    x1r = pr_ref[0, 1:2, :]
    y2r = pr_ref[0, 2:3, :]
    x2r = pr_ref[0, 3:4, :]
    pr = pr_ref[0, 4:5, :]
    y1c = pc_ref[0, 0]
    x1c = pc_ref[0, 1]
    y2c = pc_ref[0, 2]
    x2c = pc_ref[0, 3]
    pc = pc_ref[0, 4]

    posr = lax.broadcasted_iota(jnp.int32, (1, KP), 1)
    posc = lax.broadcasted_iota(jnp.int32, (KP, 1), 0)
    validr = posr < K
    validc = posc < K
    arear = (y2r - y1r) * (x2r - x1r)
    areac = (y2c - y1c) * (x2c - x1c)

    nt = KP // _TILE
    for t in range(nt):
        sl = slice(t * _TILE, (t + 1) * _TILE)
        y1s, x1s, y2s, x2s = y1c[sl], x1c[sl], y2c[sl], x2c[sl]
        pcs, poss, areas, vals = pc[sl], posc[sl], areac[sl], validc[sl]
        iy1 = jnp.maximum(y1s, y1r)
        ix1 = jnp.maximum(x1s, x1r)
        iy2 = jnp.minimum(y2s, y2r)
        ix2 = jnp.minimum(x2s, x2r)
        inter = jnp.maximum(iy2 - iy1, 0.0) * jnp.maximum(ix2 - ix1, 0.0)
        iou = inter / (areas + arear - inter + jnp.float32(1e-9))
        beats = ((pcs > pr) | ((pcs == pr) & (poss < posr))) & vals & validr
        s_ref[sl, :] = (beats & (iou > jnp.float32(IOU_T))).astype(jnp.bfloat16)

    validf = validr.astype(jnp.float32)
    smat = s_ref[...]

    def w_cond(carry):
        return carry[1]

    def w_body(carry):
        kp, _ = carry
        supp = lax.dot_general(kp.astype(jnp.bfloat16), smat,
                               (((1,), (0,)), ((), ())),
                               preferred_element_type=jnp.float32)
        new = jnp.where(validr & (supp == 0.0), 1.0, 0.0).astype(jnp.float32)
        return new, jnp.any(new != kp)

    keep, _ = lax.while_loop(w_cond, w_body, (validf, True))

    for t in range(nt):
        sl = slice(t * _TILE, (t + 1) * _TILE)
        pcs, poss, vals = pc[sl], posc[sl], validc[sl]
        beats = ((pcs > pr) | ((pcs == pr) & (poss < posr))) & vals & validr
        s_ref[sl, :] = beats.astype(jnp.bfloat16)

    rank = lax.dot_general(keep.astype(jnp.bfloat16), s_ref[...],
                           (((1,), (0,)), ((), ())),
                           preferred_element_type=jnp.float32)
    keptrank = jnp.where(keep > 0.0, rank, jnp.float32(1e9))

    srow = lax.broadcasted_iota(jnp.int32, (OUTP, 1), 0).astype(jnp.float32)
    out5 = jnp.zeros((OUTP, 5), jnp.float32)
    for t in range(nt):
        sl = slice(t * _TILE, (t + 1) * _TILE)
        krs = keptrank[:, sl]
        pseg = (krs == srow).astype(jnp.float32)
        bseg = jnp.concatenate(
            [y1c[sl], x1c[sl], y2c[sl], x2c[sl], pc[sl]], axis=1)
        out5 = out5 + lax.dot_general(pseg, bseg,
                                      (((1,), (0,)), ((), ())),
                                      preferred_element_type=jnp.float32,
                                      precision=lax.Precision.HIGHEST)
    boxes_ref[0] = out5[:, 0:4]
    scores_ref[0] = out5[:, 4:5]


def _run_nms(gath_r, gath_c):
    return pl.pallas_call(
        _nms_body,
        grid=(B,),
        in_specs=[
            pl.BlockSpec((1, 5, KP), lambda b: (b, 0, 0)),
            pl.BlockSpec((1, 5, KP, 1), lambda b: (b, 0, 0, 0)),
        ],
        out_specs=(
            pl.BlockSpec((1, OUTP, 4), lambda b: (b, 0, 0)),
            pl.BlockSpec((1, OUTP, 1), lambda b: (b, 0, 0)),
        ),
        out_shape=(
            jax.ShapeDtypeStruct((B, OUTP, 4), jnp.float32),
            jax.ShapeDtypeStruct((B, OUTP, 1), jnp.float32),
        ),
        scratch_shapes=[pltpu.VMEM((KP, KP), jnp.bfloat16)],
    )(gath_r, gath_c)


@jax.jit
def kernel(rpn_bbox_deltas, rpn_labels, anchors):
    pad = AP - A
    labels_p = jnp.pad(rpn_labels, ((0, 0), (0, pad)),
                       constant_values=NEG_INF)
    deltas_p = jnp.pad(rpn_bbox_deltas, ((0, 0), (0, pad), (0, 0)))
    deltas_t = jnp.transpose(deltas_p, (0, 2, 1))
    anchors_t = jnp.transpose(jnp.pad(anchors, ((0, pad), (0, 0))), (1, 0))

    key, aux, md = _run_select(labels_p)
    planes = _run_decode(deltas_t, labels_p, anchors_t, md)
    planes_flat = planes.reshape(5, B * AP)
    gath = _run_sc(key, aux, planes_flat)
    gath_t = jnp.transpose(gath.reshape(5, B, KP), (1, 0, 2))
    gath_r = gath_t
    gath_c = gath_t.reshape(B, 5, KP, 1)
    boxes, scores = _run_nms(gath_r, gath_c)
    return boxes[:, :OUTN, :], scores[:, :OUTN, 0]


# speedup vs baseline: 26.5258x; 1.5961x over previous
import functools

import jax
import jax.numpy as jnp
from jax import lax
from jax.experimental import pallas as pl
from jax.experimental.pallas import tpu as pltpu
from jax.experimental.pallas import tpu_sc as plsc

B = 8
A = 20000
AP = 20480
K = 2000
KP = 2048
OUTN = 300
OUTP = 384
IOU_T = 0.7
NEG_INF = float("-inf")


def _select_body(lab_ref, dest_ref, md_ref):
    x = lab_ref[...]
    m = jnp.max(x, axis=1, keepdims=True)
    u = jnp.exp(x - m)
    d = jnp.sum(u, axis=1, keepdims=True)
    p = u / d
    key = lax.bitcast_convert_type(p, jnp.int32)
    col = lax.broadcasted_iota(jnp.int32, (B, AP), 1)
    key = jnp.where(col < A, key, -1)

    def bit_step(k, t):
        bit = 30 - k
        cand = t | jnp.left_shift(jnp.int32(1), bit)
        cnt = jnp.sum((key >= cand).astype(jnp.int32), axis=1, keepdims=True)
        return jnp.where(cnt >= K, cand, t)

    t = lax.fori_loop(0, 31, bit_step, jnp.zeros((B, 1), jnp.int32))
    c1 = jnp.sum((key > t).astype(jnp.int32), axis=1, keepdims=True)
    k2 = K - c1
    gt = key > t
    eq = key == t
    packed = gt.astype(jnp.int32) * 32768 + eq.astype(jnp.int32)
    c = packed
    sh = 1
    while sh < AP:
        z = jnp.zeros((B, sh), jnp.int32)
        c = c + jnp.concatenate([z, c[:, : AP - sh]], axis=1)
        sh *= 2
    pref = c - packed
    pg = lax.shift_right_logical(pref, 15)
    pe = pref & 32767
    sel = gt | (eq & (pe < k2))
    dest_ref[...] = jnp.where(sel, pg + jnp.minimum(pe, k2), -1)
    lane = lax.broadcasted_iota(jnp.int32, (B, 128), 1)
    md_ref[...] = jnp.where(lane == 0, m, jnp.where(lane == 1, d, 0.0))


def _run_select(labels_p):
    return pl.pallas_call(
        _select_body,
        out_shape=(
            jax.ShapeDtypeStruct((B, AP), jnp.int32),
            jax.ShapeDtypeStruct((B, 128), jnp.float32),
        ),
    )(labels_p)


_CH = 2560


def _decode_body(dt_ref, lab_ref, at_ref, md_ref, out_ref):
    x = lab_ref[...]
    m = md_ref[:, 0:1]
    den = md_ref[:, 1:2]
    p = jnp.exp(x - m) / den
    d0 = dt_ref[:, 0, :] * jnp.float32(0.1)
    d1 = dt_ref[:, 1, :] * jnp.float32(0.1)
    d2 = dt_ref[:, 2, :] * jnp.float32(0.2)
    d3 = dt_ref[:, 3, :] * jnp.float32(0.2)
    a0 = at_ref[0:1, :]
    a1 = at_ref[1:2, :]
    a2 = at_ref[2:3, :]
    a3 = at_ref[3:4, :]
    w = a3 - a1
    h = a2 - a0
    cx = a1 + 0.5 * w
    cy = a0 + 0.5 * h
    bw = jnp.exp(d3) * w
    bh = jnp.exp(d2) * h
    bcx = d1 * w + cx
    bcy = d0 * h + cy
    y1 = bcy - 0.5 * bh
    x1 = bcx - 0.5 * bw
    y2 = bh + y1
    x2 = bw + x1
    out_ref[0] = y1
    out_ref[1] = x1
    out_ref[2] = y2
    out_ref[3] = x2
    out_ref[4] = p


def _run_decode(deltas_t, labels_p, anchors_t, md):
    n_ch = AP // _CH
    return pl.pallas_call(
        _decode_body,
        grid=(n_ch,),
        in_specs=[
            pl.BlockSpec((B, 4, _CH), lambda i: (0, 0, i)),
            pl.BlockSpec((B, _CH), lambda i: (0, i)),
            pl.BlockSpec((4, _CH), lambda i: (0, i)),
            pl.BlockSpec((B, 128), lambda i: (0, 0)),
        ],
        out_specs=pl.BlockSpec((5, B, _CH), lambda i: (0, 0, i)),
        out_shape=jax.ShapeDtypeStruct((5, B, AP), jnp.float32),
    )(deltas_t, labels_p, anchors_t, md)


_NPB = 4
_SEG = AP // _NPB
_NCHUNK = _SEG // 16
_ROWW = KP + 8
_NSL = KP // _NPB


def _sc_body(dest_hbm, p0_hbm, p1_hbm, p2_hbm, p3_hbm, p4_hbm, gath_hbm,
             dest_v, idx_buf, aidx_v, gidx_v, r0, r1, r2, r3, r4,
             merged_sp, sem):
    c = lax.axis_index("c")
    s = lax.axis_index("s")
    b = c * 4 + s // _NPB
    bc = s // _NPB
    q = s % _NPB
    base = q * _SEG
    row0 = bc * KP

    pltpu.sync_copy(dest_hbm.at[pl.ds(b * AP + base, _SEG)], dest_v)
    lane = lax.iota(jnp.int32, 16)
    zero16 = jnp.zeros((16,), jnp.int32)
    gbase = b * AP + base + 1

    def zero_step(j, _):
        idx_buf[pl.ds(j * 16, 16)] = zero16
        aidx_v[pl.ds(j * 16, 16)] = lane + (row0 + j * 16)
        return 0

    lax.fori_loop(0, KP // 16, zero_step, 0)

    @pl.when(q == 0)
    def _():
        pltpu.sync_copy(idx_buf, merged_sp.at[pl.ds(row0, KP)])

    plsc.subcore_barrier()

    def scat_step(i, _):
        d = dest_v[pl.ds(i * 16, 16)]
        sel = d >= 0
        idx = jnp.maximum(d, 0)
        vals = lane + (gbase + i * 16)
        plsc.store_scatter(idx_buf, [idx], vals, mask=sel)
        return 0

    lax.fori_loop(0, _NCHUNK, scat_step, 0)
    pltpu.sync_copy(idx_buf, merged_sp.at[aidx_v], add=True)
    plsc.subcore_barrier()

    pltpu.sync_copy(merged_sp.at[pl.ds(row0 + q * _NSL, _NSL)], gidx_v)

    def adj_step(j, _):
        v = gidx_v[pl.ds(j * 16, 16)]
        gidx_v[pl.ds(j * 16, 16)] = jnp.maximum(v - 1, 0)
        return 0

    lax.fori_loop(0, _NSL // 16, adj_step, 0)
    off = b * KP + q * _NSL
    rows = (r0, r1, r2, r3, r4)
    copies = []
    for ci, plane in enumerate((p0_hbm, p1_hbm, p2_hbm, p3_hbm, p4_hbm)):
        copies.append(pltpu.async_copy(plane.at[gidx_v], rows[ci], sem))
    for cp in copies:
        cp.wait()
    for ci in range(5):
        pltpu.sync_copy(rows[ci],
                        gath_hbm.at[pl.ds(ci * (B * KP) + off, _NSL)])


def _run_sc(dest, planes_flat):
    mesh = plsc.VectorSubcoreMesh(core_axis_name="c", subcore_axis_name="s")
    fn = pl.kernel(
        _sc_body,
        out_type=jax.ShapeDtypeStruct((5 * B * KP,), jnp.float32),
        mesh=mesh,
        scratch_types=[
            pltpu.VMEM((_SEG,), jnp.int32),
            pltpu.VMEM((KP,), jnp.int32),
            pltpu.VMEM((KP,), jnp.int32),
            pltpu.VMEM((_NSL,), jnp.int32),
            pltpu.VMEM((_NSL,), jnp.float32),
            pltpu.VMEM((_NSL,), jnp.float32),
            pltpu.VMEM((_NSL,), jnp.float32),
            pltpu.VMEM((_NSL,), jnp.float32),
            pltpu.VMEM((_NSL,), jnp.float32),
            pltpu.VMEM_SHARED((4 * KP,), jnp.int32),
            pltpu.SemaphoreType.DMA,
        ],
        compiler_params=pltpu.CompilerParams(needs_layout_passes=False),
    )
    return fn(dest.reshape(-1), planes_flat[0], planes_flat[1],
              planes_flat[2], planes_flat[3], planes_flat[4])


_TILE = 256


def _nms_body(pr_ref, pc_ref, boxes_ref, scores_ref, s_ref, b_ref):
    y1r = pr_ref[0, 0:1, :]
    x1r = pr_ref[0, 1:2, :]
    y2r = pr_ref[0, 2:3, :]
    x2r = pr_ref[0, 3:4, :]
    pr = pr_ref[0, 4:5, :]
    y1c = pc_ref[0, 0]
    x1c = pc_ref[0, 1]
    y2c = pc_ref[0, 2]
    x2c = pc_ref[0, 3]
    pc = pc_ref[0, 4]

    posr = lax.broadcasted_iota(jnp.int32, (1, KP), 1)
    posc = lax.broadcasted_iota(jnp.int32, (KP, 1), 0)
    validr = posr < K
    validc = posc < K
    arear = (y2r - y1r) * (x2r - x1r)
    areac = (y2c - y1c) * (x2c - x1c)

    nt = KP // _TILE
    for t in range(nt):
        sl = slice(t * _TILE, (t + 1) * _TILE)
        y1s, x1s, y2s, x2s = y1c[sl], x1c[sl], y2c[sl], x2c[sl]
        pcs, poss, areas, vals = pc[sl], posc[sl], areac[sl], validc[sl]
        iy1 = jnp.maximum(y1s, y1r)
        ix1 = jnp.maximum(x1s, x1r)
        iy2 = jnp.minimum(y2s, y2r)
        ix2 = jnp.minimum(x2s, x2r)
        inter = jnp.maximum(iy2 - iy1, 0.0) * jnp.maximum(ix2 - ix1, 0.0)
        iou = inter / (areas + arear - inter + jnp.float32(1e-9))
        beats = ((pcs > pr) | ((pcs == pr) & (poss < posr))) & vals & validr
        b_ref[sl, :] = beats.astype(jnp.bfloat16)
        s_ref[sl, :] = (beats & (iou > jnp.float32(IOU_T))).astype(jnp.bfloat16)

    validf = validr.astype(jnp.float32)
    smat = s_ref[...]

    def w_cond(carry):
        return carry[1]

    def w_body(carry):
        kp, _ = carry
        supp = lax.dot_general(kp.astype(jnp.bfloat16), smat,
                               (((1,), (0,)), ((), ())),
                               preferred_element_type=jnp.float32)
        new = jnp.where(validr & (supp == 0.0), 1.0, 0.0).astype(jnp.float32)
        return new, jnp.any(new != kp)

    keep, _ = lax.while_loop(w_cond, w_body, (validf, True))

    rank = lax.dot_general(keep.astype(jnp.bfloat16), b_ref[...],
                           (((1,), (0,)), ((), ())),
                           preferred_element_type=jnp.float32)
    keptrank = jnp.where(keep > 0.0, rank, jnp.float32(1e9))

    srow = lax.broadcasted_iota(jnp.int32, (OUTP, 1), 0).astype(jnp.float32)
    out5 = jnp.zeros((OUTP, 5), jnp.float32)
    for t in range(nt):
        sl = slice(t * _TILE, (t + 1) * _TILE)
        krs = keptrank[:, sl]
        pseg = (krs == srow).astype(jnp.float32)
        bseg = jnp.concatenate(
            [y1c[sl], x1c[sl], y2c[sl], x2c[sl], pc[sl]], axis=1)
        out5 = out5 + lax.dot_general(pseg, bseg,
                                      (((1,), (0,)), ((), ())),
                                      preferred_element_type=jnp.float32,
                                      precision=lax.Precision.HIGHEST)
    boxes_ref[0] = out5[:, 0:4]
    scores_ref[0] = out5[:, 4:5]


def _run_nms(gath_r, gath_c):
    return pl.pallas_call(
        _nms_body,
        grid=(B,),
        in_specs=[
            pl.BlockSpec((1, 5, KP), lambda b: (b, 0, 0)),
            pl.BlockSpec((1, 5, KP, 1), lambda b: (b, 0, 0, 0)),
        ],
        out_specs=(
            pl.BlockSpec((1, OUTP, 4), lambda b: (b, 0, 0)),
            pl.BlockSpec((1, OUTP, 1), lambda b: (b, 0, 0)),
        ),
        out_shape=(
            jax.ShapeDtypeStruct((B, OUTP, 4), jnp.float32),
            jax.ShapeDtypeStruct((B, OUTP, 1), jnp.float32),
        ),
        scratch_shapes=[pltpu.VMEM((KP, KP), jnp.bfloat16),
                        pltpu.VMEM((KP, KP), jnp.bfloat16)],
    )(gath_r, gath_c)


@jax.jit
def kernel(rpn_bbox_deltas, rpn_labels, anchors):
    pad = AP - A
    labels_p = jnp.pad(rpn_labels, ((0, 0), (0, pad)),
                       constant_values=NEG_INF)
    deltas_p = jnp.pad(rpn_bbox_deltas, ((0, 0), (0, pad), (0, 0)))
    deltas_t = jnp.transpose(deltas_p, (0, 2, 1))
    anchors_t = jnp.transpose(jnp.pad(anchors, ((0, pad), (0, 0))), (1, 0))

    dest, md = _run_select(labels_p)
    planes = _run_decode(deltas_t, labels_p, anchors_t, md)
    planes_flat = planes.reshape(5, B * AP)
    gath = _run_sc(dest, planes_flat)
    gath_t = jnp.transpose(gath.reshape(5, B, KP), (1, 0, 2))
    gath_r = gath_t
    gath_c = gath_t.reshape(B, 5, KP, 1)
    boxes, scores = _run_nms(gath_r, gath_c)
    return boxes[:, :OUTN, :], scores[:, :OUTN, 0]
